# baseline (speedup 1.0000x reference)
import functools

import jax
import jax.numpy as jnp
from jax import lax
from jax.experimental import pallas as pl
from jax.experimental.pallas import tpu as pltpu
from jax.experimental.pallas import tpu_sc as plsc

B = 4096
L = 50
D = 64
D2 = D // 2
LANES = 16
SC_CORES = 2
SC_SUBCORES = 16
NW = SC_CORES * SC_SUBCORES
ROWS = 2 * B
ROWS_PER_W = ROWS // NW
CHUNK = 64

_MLP_BM = 512

V = 1000000
_TR_BN = 8192
_TR_GRID = -(-V // (4 * _TR_BN))
_TR_ROWS = _TR_GRID * _TR_BN
_T32_ROWS = _TR_ROWS * 4
_LAST_BLK = V // _TR_BN


def _transpose_body(in0, in1, in2, in3, out_ref):
    out_ref[:, 0 * D2:1 * D2] = in0[...].T
    out_ref[:, 1 * D2:2 * D2] = in1[...].T
    out_ref[:, 2 * D2:3 * D2] = in2[...].T
    out_ref[:, 3 * D2:4 * D2] = in3[...].T


def _transpose_tc(tableT, d_blk):
    specs = [
        pl.BlockSpec(
            (D2, _TR_BN),
            functools.partial(
                lambda k, i: (d_blk, jnp.minimum(4 * i + k, _LAST_BLK)), k))
        for k in range(4)
    ]
    return pl.pallas_call(
        _transpose_body,
        grid=(_TR_GRID,),
        in_specs=specs,
        out_specs=pl.BlockSpec((_TR_BN, 4 * D2), lambda i: (i, 0)),
        out_shape=jax.ShapeDtypeStruct((_TR_ROWS, 4 * D2), jnp.float32),
    )(tableT, tableT, tableT, tableT)


_W_GROUPS = [(0, 0, LANES), (16, 0, LANES), (32, 0, LANES), (34, 14, LANES)]


def _pool_sc(s1, s2, w1, w2, table):
    mesh = plsc.VectorSubcoreMesh(core_axis_name="c", subcore_axis_name="s")
    half = NW // 2
    rows_per_w = B // half

    @functools.partial(
        pl.kernel,
        out_type=jax.ShapeDtypeStruct((ROWS, D2), jnp.float32),
        mesh=mesh,
        scratch_types=[
            pltpu.VMEM((CHUNK, L), jnp.int32),
            pltpu.VMEM((CHUNK, L), jnp.float32),
            pltpu.VMEM((L, D2), jnp.float32),
            pltpu.VMEM((L, D2), jnp.float32),
            pltpu.VMEM((CHUNK, D2), jnp.float32),
            pltpu.SemaphoreType.DMA,
            pltpu.SemaphoreType.DMA,
        ],
        compiler_params=pltpu.CompilerParams(use_tc_tiling_on_sc=False),
    )
    def pool_kernel(s1_hbm, s2_hbm, w1_hbm, w2_hbm, table_hbm, out_hbm,
                    idx_v, w_v, rows0_v, rows1_v, out_v, sem0, sem1):
        wid = lax.axis_index("s") * SC_CORES + lax.axis_index("c")
        bufs = ((rows0_v, sem0), (rows1_v, sem1))

        def gather(r, buf):
            rows_v, sem = bufs[buf]
            return pltpu.make_async_copy(table_hbm.at[idx_v.at[r]], rows_v, sem)

        def accumulate(r, buf):
            rows_v, _ = bufs[buf]
            accs = [jnp.zeros((LANES,), jnp.float32) for _ in range(D2 // LANES)]
            for off, j0, j1 in _W_GROUPS:
                wvec = w_v[r, pl.ds(off, LANES)]
                for j in range(j0, j1):
                    t = off + j
                    wt = wvec[j]
                    for c in range(D2 // LANES):
                        accs[c] = accs[c] + wt * rows_v[t, pl.ds(c * LANES, LANES)]
            for c in range(D2 // LANES):
                out_v[r, pl.ds(c * LANES, LANES)] = accs[c] * (1.0 / L)

        def do_half(s_hbm, w_hbm, lwid, out_base):
            lbase = lwid * rows_per_w

            @pl.loop(0, rows_per_w, step=CHUNK)
            def _chunk(r0):
                pltpu.sync_copy(s_hbm.at[pl.ds(lbase + r0, CHUNK)], idx_v)
                pltpu.sync_copy(w_hbm.at[pl.ds(lbase + r0, CHUNK)], w_v)

                gather(0, 0).start()

                @pl.loop(0, CHUNK, step=2)
                def _row(r):
                    gather(r + 1, 1).start()
                    gather(r, 0).wait()
                    accumulate(r, 0)

                    @pl.when(r + 2 < CHUNK)
                    def _():
                        gather(r + 2, 0).start()

                    gather(r + 1, 1).wait()
                    accumulate(r + 1, 1)

                pltpu.sync_copy(out_v, out_hbm.at[pl.ds(out_base + lbase + r0, CHUNK)])

        @pl.when(wid < half)
        def _():
            do_half(s1_hbm, w1_hbm, wid, 0)

        @pl.when(wid >= half)
        def _():
            do_half(s2_hbm, w2_hbm, wid - half, B)

    return pool_kernel(s1, s2, w1, w2, table)


def _mlp_body(a1A_ref, a2A_ref, a1B_ref, a2B_ref,
              W1_ref, b1_ref, W2_ref, b2_ref, W3_ref, b3_ref, out_ref):
    a1A, a2A = a1A_ref[...], a2A_ref[...]
    a1B, a2B = a1B_ref[...], a2B_ref[...]
    comb = jnp.concatenate(
        [jnp.abs(a1A - a2A), jnp.abs(a1B - a2B), a1A * a2A, a1B * a2B],
        axis=1)
    h = jnp.dot(comb, W1_ref[...], preferred_element_type=jnp.float32) + b1_ref[...]
    h = jnp.dot(h, W2_ref[...], preferred_element_type=jnp.float32) + b2_ref[...]
    out_ref[...] = jnp.dot(h, W3_ref[...],
                           preferred_element_type=jnp.float32) + b3_ref[...]


def _mlp_tc(pooledA, pooledB, W1, b1, W2, b2, W3, b3):
    nblk = B // _MLP_BM
    h1 = W1.shape[1]
    nc = W3.shape[1]
    return pl.pallas_call(
        _mlp_body,
        grid=(nblk,),
        in_specs=[
            pl.BlockSpec((_MLP_BM, D2), lambda i: (i, 0)),
            pl.BlockSpec((_MLP_BM, D2), lambda i: (i + nblk, 0)),
            pl.BlockSpec((_MLP_BM, D2), lambda i: (i, 0)),
            pl.BlockSpec((_MLP_BM, D2), lambda i: (i + nblk, 0)),
            pl.BlockSpec((2 * D, h1), lambda i: (0, 0)),
            pl.BlockSpec((1, h1), lambda i: (0, 0)),
            pl.BlockSpec((h1, h1), lambda i: (0, 0)),
            pl.BlockSpec((1, h1), lambda i: (0, 0)),
            pl.BlockSpec((h1, nc), lambda i: (0, 0)),
            pl.BlockSpec((1, nc), lambda i: (0, 0)),
        ],
        out_specs=pl.BlockSpec((_MLP_BM, nc), lambda i: (i, 0)),
        out_shape=jax.ShapeDtypeStruct((B, nc), jnp.float32),
    )(pooledA, pooledA, pooledB, pooledB,
      W1, b1[None, :], W2, b2[None, :], W3, b3[None, :])


def kernel(s1, s2, w1, w2, table, W1, b1, W2, b2, W3, b3):
    tT = table.T
    tableA = _transpose_tc(tT, 0).reshape(_T32_ROWS, D2)
    tableB = _transpose_tc(tT, 1).reshape(_T32_ROWS, D2)

    def _remap(s):
        r = s.astype(jnp.int32)
        return ((r >> 15) << 15) + ((r & (_TR_BN - 1)) << 2) + ((r >> 13) & 3)

    i1, i2 = _remap(s1), _remap(s2)
    pooledA = _pool_sc(i1, i2, w1, w2, tableA)
    pooledB = _pool_sc(i1, i2, w1, w2, tableB)
    return _mlp_tc(pooledA, pooledB, W1, b1, W2, b2, W3, b3)


# speedup vs baseline: 1.6084x; 1.6084x over previous
import functools

import jax
import jax.numpy as jnp
from jax import lax
from jax.experimental import pallas as pl
from jax.experimental.pallas import tpu as pltpu
from jax.experimental.pallas import tpu_sc as plsc

B = 4096
L = 50
D = 64
LANES = 16
SC_CORES = 2
SC_SUBCORES = 16
NW = SC_CORES * SC_SUBCORES
ROWS = 2 * B
ROWS_PER_W = ROWS // NW
CHUNK = 64

_MLP_BM = 512


V = 1000000
_TR_BN = 8192
_TR_GRID = -(-V // (2 * _TR_BN))
_TR_ROWS = _TR_GRID * _TR_BN
_T64_ROWS = _TR_ROWS * 2


def _transpose_body(inl_ref, inr_ref, out_ref):
    out_ref[:, 0:D] = inl_ref[...].T
    out_ref[:, D:2 * D] = inr_ref[...].T


def _transpose_tc(tableT):
    return pl.pallas_call(
        _transpose_body,
        grid=(_TR_GRID,),
        in_specs=[
            pl.BlockSpec((D, _TR_BN), lambda i: (0, 2 * i)),
            pl.BlockSpec((D, _TR_BN),
                         lambda i: (0, jnp.minimum(2 * i + 1, V // _TR_BN))),
        ],
        out_specs=pl.BlockSpec((_TR_BN, 2 * D), lambda i: (i, 0)),
        out_shape=jax.ShapeDtypeStruct((_TR_ROWS, 2 * D), jnp.float32),
    )(tableT, tableT)


_W_GROUPS = [(0, 0, LANES), (16, 0, LANES), (32, 0, LANES), (34, 14, LANES)]


def _pool_sc(s1, s2, w1, w2, table):
    mesh = plsc.VectorSubcoreMesh(core_axis_name="c", subcore_axis_name="s")
    half = NW // 2
    rows_per_w = B // half

    @functools.partial(
        pl.kernel,
        out_type=jax.ShapeDtypeStruct((ROWS, D), jnp.float32),
        mesh=mesh,
        scratch_types=[
            pltpu.VMEM((CHUNK, L), jnp.int32),
            pltpu.VMEM((CHUNK, L), jnp.float32),
            pltpu.VMEM((L, D), jnp.float32),
            pltpu.VMEM((L, D), jnp.float32),
            pltpu.VMEM((CHUNK, D), jnp.float32),
            pltpu.SemaphoreType.DMA,
            pltpu.SemaphoreType.DMA,
        ],
        compiler_params=pltpu.CompilerParams(use_tc_tiling_on_sc=False),
    )
    def pool_kernel(s1_hbm, s2_hbm, w1_hbm, w2_hbm, table_hbm, out_hbm,
                    idx_v, w_v, rows0_v, rows1_v, out_v, sem0, sem1):
        wid = lax.axis_index("s") * SC_CORES + lax.axis_index("c")
        bufs = ((rows0_v, sem0), (rows1_v, sem1))

        def gather(r, buf):
            rows_v, sem = bufs[buf]
            return pltpu.make_async_copy(table_hbm.at[idx_v.at[r]], rows_v, sem)

        def accumulate(r, buf):
            rows_v, _ = bufs[buf]
            accs = [jnp.zeros((LANES,), jnp.float32) for _ in range(D // LANES)]
            for off, j0, j1 in _W_GROUPS:
                wvec = w_v[r, pl.ds(off, LANES)]
                for j in range(j0, j1):
                    t = off + j
                    wt = wvec[j]
                    for c in range(D // LANES):
                        accs[c] = accs[c] + wt * rows_v[t, pl.ds(c * LANES, LANES)]
            for c in range(D // LANES):
                out_v[r, pl.ds(c * LANES, LANES)] = accs[c] * (1.0 / L)

        def do_half(s_hbm, w_hbm, lwid, out_base):
            lbase = lwid * rows_per_w

            @pl.loop(0, rows_per_w, step=CHUNK)
            def _chunk(r0):
                pltpu.sync_copy(s_hbm.at[pl.ds(lbase + r0, CHUNK)], idx_v)
                pltpu.sync_copy(w_hbm.at[pl.ds(lbase + r0, CHUNK)], w_v)

                gather(0, 0).start()

                @pl.loop(0, CHUNK, step=2)
                def _row(r):
                    gather(r + 1, 1).start()
                    gather(r, 0).wait()
                    accumulate(r, 0)

                    @pl.when(r + 2 < CHUNK)
                    def _():
                        gather(r + 2, 0).start()

                    gather(r + 1, 1).wait()
                    accumulate(r + 1, 1)

                pltpu.sync_copy(out_v, out_hbm.at[pl.ds(out_base + lbase + r0, CHUNK)])

        @pl.when(wid < half)
        def _():
            do_half(s1_hbm, w1_hbm, wid, 0)

        @pl.when(wid >= half)
        def _():
            do_half(s2_hbm, w2_hbm, wid - half, B)

    return pool_kernel(s1, s2, w1, w2, table)


def _mlp_body(p1_ref, p2_ref, W1_ref, b1_ref, W2_ref, b2_ref, W3_ref, b3_ref,
              out_ref):
    a = p1_ref[...]
    b = p2_ref[...]
    comb = jnp.concatenate([jnp.abs(a - b), a * b], axis=1)
    h = jnp.dot(comb, W1_ref[...], preferred_element_type=jnp.float32) + b1_ref[...]
    h = jnp.dot(h, W2_ref[...], preferred_element_type=jnp.float32) + b2_ref[...]
    out_ref[...] = jnp.dot(h, W3_ref[...], preferred_element_type=jnp.float32) + b3_ref[...]


def _mlp_tc(pooled, W1, b1, W2, b2, W3, b3):
    nblk = B // _MLP_BM
    h1 = W1.shape[1]
    nc = W3.shape[1]
    return pl.pallas_call(
        _mlp_body,
        grid=(nblk,),
        in_specs=[
            pl.BlockSpec((_MLP_BM, D), lambda i: (i, 0)),
            pl.BlockSpec((_MLP_BM, D), lambda i: (i + nblk, 0)),
            pl.BlockSpec((2 * D, h1), lambda i: (0, 0)),
            pl.BlockSpec((1, h1), lambda i: (0, 0)),
            pl.BlockSpec((h1, h1), lambda i: (0, 0)),
            pl.BlockSpec((1, h1), lambda i: (0, 0)),
            pl.BlockSpec((h1, nc), lambda i: (0, 0)),
            pl.BlockSpec((1, nc), lambda i: (0, 0)),
        ],
        out_specs=pl.BlockSpec((_MLP_BM, nc), lambda i: (i, 0)),
        out_shape=jax.ShapeDtypeStruct((B, nc), jnp.float32),
    )(pooled, pooled, W1, b1[None, :], W2, b2[None, :], W3, b3[None, :])


def kernel(s1, s2, w1, w2, table, W1, b1, W2, b2, W3, b3):
    table64 = _transpose_tc(table.T).reshape(_T64_ROWS, D)

    def _remap(s):
        r = s.astype(jnp.int32)
        return ((r >> 14) << 14) + ((r & (_TR_BN - 1)) << 1) + ((r >> 13) & 1)

    pooled = _pool_sc(_remap(s1), _remap(s2), w1, w2, table64)
    return _mlp_tc(pooled, W1, b1, W2, b2, W3, b3)


# speedup vs baseline: 1.6678x; 1.0369x over previous
import functools

import jax
import jax.numpy as jnp
from jax import lax
from jax.experimental import pallas as pl
from jax.experimental.pallas import tpu as pltpu
from jax.experimental.pallas import tpu_sc as plsc

B = 4096
L = 50
D = 64
LANES = 16
SC_CORES = 2
SC_SUBCORES = 16
NW = SC_CORES * SC_SUBCORES
ROWS = 2 * B
ROWS_PER_W = ROWS // NW
CHUNK = 64

_MLP_BM = 512


V = 1000000
_TR_BN = 16384
_TR_SH = _TR_BN.bit_length() - 1
_TR_GRID = -(-V // (2 * _TR_BN))
_TR_ROWS = _TR_GRID * _TR_BN
_T64_ROWS = _TR_ROWS * 2


def _transpose_body(inl_ref, inr_ref, out_ref):
    out_ref[:, 0:D] = inl_ref[...].T
    out_ref[:, D:2 * D] = inr_ref[...].T


def _transpose_tc(tableT):
    return pl.pallas_call(
        _transpose_body,
        grid=(_TR_GRID,),
        in_specs=[
            pl.BlockSpec((D, _TR_BN), lambda i: (0, 2 * i)),
            pl.BlockSpec((D, _TR_BN),
                         lambda i: (0, jnp.minimum(2 * i + 1, V // _TR_BN))),
        ],
        out_specs=pl.BlockSpec((_TR_BN, 2 * D), lambda i: (i, 0)),
        out_shape=jax.ShapeDtypeStruct((_TR_ROWS, 2 * D), jnp.float32),
    )(tableT, tableT)


_W_GROUPS = [(0, 0, LANES), (16, 0, LANES), (32, 0, LANES), (34, 14, LANES)]


def _pool_sc(s1, s2, w1, w2, table):
    mesh = plsc.VectorSubcoreMesh(core_axis_name="c", subcore_axis_name="s")
    half = NW // 2
    rows_per_w = B // half

    @functools.partial(
        pl.kernel,
        out_type=jax.ShapeDtypeStruct((ROWS, D), jnp.float32),
        mesh=mesh,
        scratch_types=[
            pltpu.VMEM((CHUNK, L), jnp.int32),
            pltpu.VMEM((CHUNK, L), jnp.float32),
            pltpu.VMEM((L, D), jnp.float32),
            pltpu.VMEM((L, D), jnp.float32),
            pltpu.VMEM((CHUNK, D), jnp.float32),
            pltpu.SemaphoreType.DMA,
            pltpu.SemaphoreType.DMA,
        ],
        compiler_params=pltpu.CompilerParams(use_tc_tiling_on_sc=False),
    )
    def pool_kernel(s1_hbm, s2_hbm, w1_hbm, w2_hbm, table_hbm, out_hbm,
                    idx_v, w_v, rows0_v, rows1_v, out_v, sem0, sem1):
        wid = lax.axis_index("s") * SC_CORES + lax.axis_index("c")
        bufs = ((rows0_v, sem0), (rows1_v, sem1))

        def gather(r, buf):
            rows_v, sem = bufs[buf]
            return pltpu.make_async_copy(table_hbm.at[idx_v.at[r]], rows_v, sem)

        def accumulate(r, buf):
            rows_v, _ = bufs[buf]
            accs = [jnp.zeros((LANES,), jnp.float32) for _ in range(D // LANES)]
            for off, j0, j1 in _W_GROUPS:
                wvec = w_v[r, pl.ds(off, LANES)]
                for j in range(j0, j1):
                    t = off + j
                    wt = wvec[j]
                    for c in range(D // LANES):
                        accs[c] = accs[c] + wt * rows_v[t, pl.ds(c * LANES, LANES)]
            for c in range(D // LANES):
                out_v[r, pl.ds(c * LANES, LANES)] = accs[c] * (1.0 / L)

        def do_half(s_hbm, w_hbm, lwid, out_base):
            lbase = lwid * rows_per_w

            @pl.loop(0, rows_per_w, step=CHUNK)
            def _chunk(r0):
                pltpu.sync_copy(s_hbm.at[pl.ds(lbase + r0, CHUNK)], idx_v)
                pltpu.sync_copy(w_hbm.at[pl.ds(lbase + r0, CHUNK)], w_v)

                gather(0, 0).start()

                @pl.loop(0, CHUNK, step=2)
                def _row(r):
                    gather(r + 1, 1).start()
                    gather(r, 0).wait()
                    accumulate(r, 0)

                    @pl.when(r + 2 < CHUNK)
                    def _():
                        gather(r + 2, 0).start()

                    gather(r + 1, 1).wait()
                    accumulate(r + 1, 1)

                pltpu.sync_copy(out_v, out_hbm.at[pl.ds(out_base + lbase + r0, CHUNK)])

        @pl.when(wid < half)
        def _():
            do_half(s1_hbm, w1_hbm, wid, 0)

        @pl.when(wid >= half)
        def _():
            do_half(s2_hbm, w2_hbm, wid - half, B)

    return pool_kernel(s1, s2, w1, w2, table)


def _mlp_body(p1_ref, p2_ref, W1_ref, b1_ref, W2_ref, b2_ref, W3_ref, b3_ref,
              out_ref):
    a = p1_ref[...]
    b = p2_ref[...]
    comb = jnp.concatenate([jnp.abs(a - b), a * b], axis=1)
    h = jnp.dot(comb, W1_ref[...], preferred_element_type=jnp.float32) + b1_ref[...]
    h = jnp.dot(h, W2_ref[...], preferred_element_type=jnp.float32) + b2_ref[...]
    out_ref[...] = jnp.dot(h, W3_ref[...], preferred_element_type=jnp.float32) + b3_ref[...]


def _mlp_tc(pooled, W1, b1, W2, b2, W3, b3):
    nblk = B // _MLP_BM
    h1 = W1.shape[1]
    nc = W3.shape[1]
    return pl.pallas_call(
        _mlp_body,
        grid=(nblk,),
        in_specs=[
            pl.BlockSpec((_MLP_BM, D), lambda i: (i, 0)),
            pl.BlockSpec((_MLP_BM, D), lambda i: (i + nblk, 0)),
            pl.BlockSpec((2 * D, h1), lambda i: (0, 0)),
            pl.BlockSpec((1, h1), lambda i: (0, 0)),
            pl.BlockSpec((h1, h1), lambda i: (0, 0)),
            pl.BlockSpec((1, h1), lambda i: (0, 0)),
            pl.BlockSpec((h1, nc), lambda i: (0, 0)),
            pl.BlockSpec((1, nc), lambda i: (0, 0)),
        ],
        out_specs=pl.BlockSpec((_MLP_BM, nc), lambda i: (i, 0)),
        out_shape=jax.ShapeDtypeStruct((B, nc), jnp.float32),
    )(pooled, pooled, W1, b1[None, :], W2, b2[None, :], W3, b3[None, :])


def kernel(s1, s2, w1, w2, table, W1, b1, W2, b2, W3, b3):
    table64 = _transpose_tc(table.T).reshape(_T64_ROWS, D)

    def _remap(s):
        r = s.astype(jnp.int32)
        return (((r >> (_TR_SH + 1)) << (_TR_SH + 1))
                + ((r & (_TR_BN - 1)) << 1) + ((r >> _TR_SH) & 1))

    pooled = _pool_sc(_remap(s1), _remap(s2), w1, w2, table64)
    return _mlp_tc(pooled, W1, b1, W2, b2, W3, b3)
